# baseline (speedup 1.0000x reference)
import functools

import jax
import jax.numpy as jnp
from jax import lax
from jax.experimental import pallas as pl
from jax.experimental.pallas import tpu as pltpu
from jax.experimental.pallas import tpu_sc as plsc

N = 10000
E = 320000
DF = 128
DE = 16
EMB = 128
NLAYER = 5
G = 64
NT = 1

NC = 2
NS = 16
NW = NC * NS
C = 128
NCHUNK = 79
EW = NCHUNK * C
EPAD = NW * EW
NCHUNKW = 80
CAPW = NCHUNKW * C
NPAD = 10112
RPT = NPAD // NS


BE = 2048


def _edge_emb_body(ea_ref, w_ref, o_ref):
    r = jnp.dot(ea_ref[...], w_ref[...], preferred_element_type=jnp.float32)
    for l in range(NLAYER):
        o_ref[l] = r[:, l * EMB:(l + 1) * EMB]


def _edge_emb(ea_pad, w_cat):
    return pl.pallas_call(
        _edge_emb_body,
        grid=(EPAD // BE,),
        in_specs=[
            pl.BlockSpec((BE, DE), lambda i: (i, 0)),
            pl.BlockSpec((DE, NLAYER * EMB), lambda i: (0, 0)),
        ],
        out_specs=pl.BlockSpec((NLAYER, BE, EMB), lambda i: (0, i, 0)),
        out_shape=jax.ShapeDtypeStruct((NLAYER, EPAD, EMB), jnp.float32),
    )(ea_pad, w_cat)


def _encoder_body(x_ref, w_ref, b_ref, o_ref):
    o_ref[...] = (
        jnp.dot(x_ref[...], w_ref[...], preferred_element_type=jnp.float32)
        + b_ref[...]
    )


def _encoder(x, w, b):
    return pl.pallas_call(
        _encoder_body,
        out_shape=jax.ShapeDtypeStruct((N, EMB), jnp.float32),
    )(x, w, b)


_SC_MESH = plsc.VectorSubcoreMesh(
    core_axis_name="c", subcore_axis_name="s", num_cores=NC, num_subcores=NS
)


def _make_sc_aggr(layer):
    def body(h_hbm, e_hbm, sd_hbm, z_hbm, out_hbm,
             sd_v, ev_v, hv_v, acc_sh, sem):
        c = lax.axis_index("c")
        s = lax.axis_index("s")
        wid = s * NC + c
        pltpu.sync_copy(z_hbm, acc_sh.at[pl.ds(s * RPT, RPT)])
        plsc.subcore_barrier()

        def step(j, carry):
            pltpu.sync_copy(sd_hbm.at[wid, j], sd_v)
            pltpu.async_copy(e_hbm.at[layer].at[sd_v.at[2]], ev_v, sem).wait()
            pltpu.async_copy(h_hbm.at[sd_v.at[0]], hv_v, sem).wait()

            def row(r, carry2):
                for k in range(EMB // 16):
                    sl = pl.ds(k * 16, 16)
                    ev_v[r, sl] = jnp.maximum(ev_v[r, sl] + hv_v[r, sl], 0.0)
                return carry2

            lax.fori_loop(0, C, row, 0, unroll=4)
            pltpu.sync_copy(ev_v, acc_sh.at[sd_v.at[1]], add=True)
            return carry

        lax.fori_loop(0, NCHUNKW, step, 0)
        plsc.subcore_barrier()
        pltpu.sync_copy(
            acc_sh.at[pl.ds(s * RPT, RPT)],
            out_hbm.at[c, pl.ds(s * RPT, RPT)],
        )

    return pl.kernel(
        body,
        out_type=jax.ShapeDtypeStruct((NC, NPAD, EMB), jnp.float32),
        mesh=_SC_MESH,
        scratch_types=[
            pltpu.VMEM((3, C), jnp.int32),
            pltpu.VMEM((C, EMB), jnp.float32),
            pltpu.VMEM((C, EMB), jnp.float32),
            pltpu.VMEM_SHARED((NPAD, EMB), jnp.float32),
            pltpu.SemaphoreType.DMA,
        ],
    )


_SC_AGGR = [_make_sc_aggr(l) for l in range(NLAYER)]


def _mean0(t):
    return jnp.sum(t, axis=0, keepdims=True) * (1.0 / t.shape[0])


def _var0(t, m):
    c = (t - m) ** 2
    half = c.shape[0] // 2
    s = (jnp.sum(c[:half], axis=0, keepdims=True)
         + jnp.sum(c[half:], axis=0, keepdims=True))
    return s * (1.0 / c.shape[0])


def _mlp_core(h_ref, ag_ref, w1_ref, b1_ref, g1_ref, be1_ref,
              w2_ref, b2_ref, g2_ref, be2_ref):
    pre = h_ref[...] + ag_ref[0, :N, :] + ag_ref[1, :N, :]
    t = jnp.dot(pre, w1_ref[...], preferred_element_type=jnp.float32) + b1_ref[...]
    m = _mean0(t)
    v = _var0(t, m)
    t = jnp.maximum((t - m) * lax.rsqrt(v + 1e-5) * g1_ref[...] + be1_ref[...], 0.0)
    o = jnp.dot(t, w2_ref[...], preferred_element_type=jnp.float32) + b2_ref[...]
    m2 = _mean0(o)
    v2 = _var0(o, m2)
    return (o - m2) * lax.rsqrt(v2 + 1e-5) * g2_ref[...] + be2_ref[...]


def _dense_body(h_ref, ag_ref, w1_ref, b1_ref, g1_ref, be1_ref,
                w2_ref, b2_ref, g2_ref, be2_ref, o_ref):
    o = _mlp_core(h_ref, ag_ref, w1_ref, b1_ref, g1_ref, be1_ref,
                  w2_ref, b2_ref, g2_ref, be2_ref)
    o_ref[...] = jnp.maximum(o, 0.0)


_dense = pl.pallas_call(
    _dense_body,
    out_shape=jax.ShapeDtypeStruct((N, EMB), jnp.float32),
)


def _final_body(h_ref, ag_ref, w1_ref, b1_ref, g1_ref, be1_ref,
                w2_ref, b2_ref, g2_ref, be2_ref,
                batch_ref, wp_ref, bp_ref, o_ref):
    hfin = _mlp_core(h_ref, ag_ref, w1_ref, b1_ref, g1_ref, be1_ref,
                     w2_ref, b2_ref, g2_ref, be2_ref)
    bt = batch_ref[...]
    oh = (bt == lax.broadcasted_iota(jnp.int32, (G, N), 0)).astype(jnp.float32)
    cnt = jnp.maximum(jnp.sum(oh, axis=1, keepdims=True), 1.0)
    pooled = jnp.dot(oh, hfin, preferred_element_type=jnp.float32, precision=lax.Precision.HIGHEST) / cnt
    o_ref[...] = (
        jnp.dot(pooled, wp_ref[...], preferred_element_type=jnp.float32)
        + bp_ref[...]
    )


_final = pl.pallas_call(
    _final_body,
    out_shape=jax.ShapeDtypeStruct((G, NT), jnp.float32),
)


def kernel(x, edge_index, edge_attr, batch, W_node, b_node, W_edge,
           W1, b1, g1, be1, W2, b2, g2, be2, W_pred, b_pred):
    src = edge_index[0]
    dst = edge_index[1]
    pad = EPAD - E
    order = jnp.argsort(dst, stable=True).astype(jnp.int32)
    dst_s = dst[order]
    src_s = src[order]
    bvals = dst_s[(E // NW) * jnp.arange(1, NW)]
    wkr = jnp.searchsorted(bvals, dst_s, side="right").astype(jnp.int32)
    wstart = jnp.searchsorted(wkr, jnp.arange(NW, dtype=jnp.int32),
                              side="left").astype(jnp.int32)
    flat = wkr * CAPW + (jnp.arange(E, dtype=jnp.int32) - wstart[wkr])
    srcw = jnp.zeros((NW * CAPW,), jnp.int32).at[flat].set(src_s)
    dstw = jnp.full((NW * CAPW,), N, jnp.int32).at[flat].set(dst_s)
    eidw = jnp.full((NW * CAPW,), E, jnp.int32).at[flat].set(order)
    sdw = jnp.stack([srcw.reshape(NW, NCHUNKW, C), dstw.reshape(NW, NCHUNKW, C),
                     eidw.reshape(NW, NCHUNKW, C)], axis=2)
    ea_pad = jnp.concatenate([edge_attr, jnp.zeros((pad, DE), jnp.float32)], axis=0)
    w_cat = jnp.transpose(W_edge, (1, 0, 2)).reshape(DE, NLAYER * EMB)

    e_all = _edge_emb(ea_pad, w_cat)
    z = jnp.zeros((RPT, EMB), jnp.float32)
    h = _encoder(x, W_node, b_node.reshape(1, EMB))

    for l in range(NLAYER):
        ag = _SC_AGGR[l](h, e_all, sdw, z)
        args = (h, ag, W1[l], b1[l].reshape(1, -1), g1[l].reshape(1, -1),
                be1[l].reshape(1, -1), W2[l], b2[l].reshape(1, -1),
                g2[l].reshape(1, -1), be2[l].reshape(1, -1))
        if l < NLAYER - 1:
            h = _dense(*args)
        else:
            out = _final(*args, batch.reshape(1, N), W_pred,
                         b_pred.reshape(1, NT))
    return out


# speedup vs baseline: 1.6101x; 1.6101x over previous
import functools

import jax
import jax.numpy as jnp
from jax import lax
from jax.experimental import pallas as pl
from jax.experimental.pallas import tpu as pltpu
from jax.experimental.pallas import tpu_sc as plsc

N = 10000
E = 320000
DF = 128
DE = 16
EMB = 128
NLAYER = 5
G = 64
NT = 1

NC = 2
NS = 16
NW = NC * NS
C = 128
NCHUNK = 79
EW = NCHUNK * C
EPAD = NW * EW
NCHUNKW = 80
CAPW = NCHUNKW * C
NPAD = 10112
RPT = NPAD // NS


BE = 2048


def _edge_emb_body(ea_ref, w_ref, o_ref):
    r = jnp.dot(ea_ref[...], w_ref[...], preferred_element_type=jnp.float32)
    for l in range(NLAYER):
        o_ref[l] = r[:, l * EMB:(l + 1) * EMB]


def _edge_emb(ea_pad, w_cat):
    return pl.pallas_call(
        _edge_emb_body,
        grid=(EPAD // BE,),
        in_specs=[
            pl.BlockSpec((BE, DE), lambda i: (i, 0)),
            pl.BlockSpec((DE, NLAYER * EMB), lambda i: (0, 0)),
        ],
        out_specs=pl.BlockSpec((NLAYER, BE, EMB), lambda i: (0, i, 0)),
        out_shape=jax.ShapeDtypeStruct((NLAYER, EPAD, EMB), jnp.float32),
    )(ea_pad, w_cat)


def _encoder_body(x_ref, w_ref, b_ref, o_ref):
    o_ref[...] = (
        jnp.dot(x_ref[...], w_ref[...], preferred_element_type=jnp.float32)
        + b_ref[...]
    )


def _encoder(x, w, b):
    return pl.pallas_call(
        _encoder_body,
        out_shape=jax.ShapeDtypeStruct((N, EMB), jnp.float32),
    )(x, w, b)


_SC_MESH = plsc.VectorSubcoreMesh(
    core_axis_name="c", subcore_axis_name="s", num_cores=NC, num_subcores=NS
)


def _make_sc_aggr(layer):
    def body(h_hbm, e_hbm, sd_hbm, z_hbm, out_hbm,
             sd_v, ev_v, hv_v, acc_sh, sem, sem_e):
        c = lax.axis_index("c")
        s = lax.axis_index("s")
        wid = s * NC + c
        pltpu.sync_copy(z_hbm, acc_sh.at[pl.ds(s * RPT, RPT)])
        plsc.subcore_barrier()

        pltpu.sync_copy(sd_hbm.at[wid, 0], sd_v.at[0])
        pltpu.async_copy(e_hbm.at[layer].at[sd_v.at[0].at[2]], ev_v.at[0], sem_e)

        def step(jj, carry):
            for b in range(2):
                j = jj * 2 + b
                nb = 1 - b

                @pl.when(j + 1 < NCHUNKW)
                def _():
                    pltpu.sync_copy(sd_hbm.at[wid, j + 1], sd_v.at[nb])
                    pltpu.async_copy(
                        e_hbm.at[layer].at[sd_v.at[nb].at[2]], ev_v.at[nb], sem_e)

                pltpu.async_copy(h_hbm.at[sd_v.at[b].at[0]], hv_v, sem).wait()
                pltpu.make_async_copy(
                    e_hbm.at[layer, pl.ds(0, C)], ev_v.at[b], sem_e).wait()

                def row(r, carry2):
                    for k in range(EMB // 16):
                        sl = pl.ds(k * 16, 16)
                        ev_v[b, r, sl] = jnp.maximum(
                            ev_v[b, r, sl] + hv_v[r, sl], 0.0)
                    return carry2

                lax.fori_loop(0, C, row, 0, unroll=4)
                pltpu.sync_copy(ev_v.at[b], acc_sh.at[sd_v.at[b].at[1]], add=True)
            return carry

        lax.fori_loop(0, NCHUNKW // 2, step, 0)
        plsc.subcore_barrier()
        pltpu.sync_copy(
            acc_sh.at[pl.ds(s * RPT, RPT)],
            out_hbm.at[c, pl.ds(s * RPT, RPT)],
        )

    return pl.kernel(
        body,
        out_type=jax.ShapeDtypeStruct((NC, NPAD, EMB), jnp.float32),
        mesh=_SC_MESH,
        scratch_types=[
            pltpu.VMEM((2, 3, C), jnp.int32),
            pltpu.VMEM((2, C, EMB), jnp.float32),
            pltpu.VMEM((C, EMB), jnp.float32),
            pltpu.VMEM_SHARED((NPAD, EMB), jnp.float32),
            pltpu.SemaphoreType.DMA,
            pltpu.SemaphoreType.DMA,
        ],
    )


_SC_AGGR = [_make_sc_aggr(l) for l in range(NLAYER)]


def _mean0(t):
    return jnp.sum(t, axis=0, keepdims=True) * (1.0 / t.shape[0])


def _var0(t, m):
    c = (t - m) ** 2
    half = c.shape[0] // 2
    s = (jnp.sum(c[:half], axis=0, keepdims=True)
         + jnp.sum(c[half:], axis=0, keepdims=True))
    return s * (1.0 / c.shape[0])


def _mlp_core(h_ref, ag_ref, w1_ref, b1_ref, g1_ref, be1_ref,
              w2_ref, b2_ref, g2_ref, be2_ref):
    pre = h_ref[...] + ag_ref[0, :N, :] + ag_ref[1, :N, :]
    t = jnp.dot(pre, w1_ref[...], preferred_element_type=jnp.float32) + b1_ref[...]
    m = _mean0(t)
    v = _var0(t, m)
    t = jnp.maximum((t - m) * lax.rsqrt(v + 1e-5) * g1_ref[...] + be1_ref[...], 0.0)
    o = jnp.dot(t, w2_ref[...], preferred_element_type=jnp.float32) + b2_ref[...]
    m2 = _mean0(o)
    v2 = _var0(o, m2)
    return (o - m2) * lax.rsqrt(v2 + 1e-5) * g2_ref[...] + be2_ref[...]


def _dense_body(h_ref, ag_ref, w1_ref, b1_ref, g1_ref, be1_ref,
                w2_ref, b2_ref, g2_ref, be2_ref, o_ref):
    o = _mlp_core(h_ref, ag_ref, w1_ref, b1_ref, g1_ref, be1_ref,
                  w2_ref, b2_ref, g2_ref, be2_ref)
    o_ref[...] = jnp.maximum(o, 0.0)


_dense = pl.pallas_call(
    _dense_body,
    out_shape=jax.ShapeDtypeStruct((N, EMB), jnp.float32),
)


def _final_body(h_ref, ag_ref, w1_ref, b1_ref, g1_ref, be1_ref,
                w2_ref, b2_ref, g2_ref, be2_ref,
                batch_ref, wp_ref, bp_ref, o_ref):
    hfin = _mlp_core(h_ref, ag_ref, w1_ref, b1_ref, g1_ref, be1_ref,
                     w2_ref, b2_ref, g2_ref, be2_ref)
    bt = batch_ref[...]
    oh = (bt == lax.broadcasted_iota(jnp.int32, (G, N), 0)).astype(jnp.float32)
    cnt = jnp.maximum(jnp.sum(oh, axis=1, keepdims=True), 1.0)
    pooled = jnp.dot(oh, hfin, preferred_element_type=jnp.float32, precision=lax.Precision.HIGHEST) / cnt
    o_ref[...] = (
        jnp.dot(pooled, wp_ref[...], preferred_element_type=jnp.float32)
        + bp_ref[...]
    )


_final = pl.pallas_call(
    _final_body,
    out_shape=jax.ShapeDtypeStruct((G, NT), jnp.float32),
)


def kernel(x, edge_index, edge_attr, batch, W_node, b_node, W_edge,
           W1, b1, g1, be1, W2, b2, g2, be2, W_pred, b_pred):
    src = edge_index[0]
    dst = edge_index[1]
    pad = EPAD - E
    order = jnp.argsort(dst, stable=True).astype(jnp.int32)
    dst_s = dst[order]
    bvals = dst_s[(E // NW) * jnp.arange(1, NW)]
    wstart = jnp.concatenate([
        jnp.zeros((1,), jnp.int32),
        jnp.searchsorted(dst_s, bvals, side="left").astype(jnp.int32),
        jnp.full((1,), E, jnp.int32),
    ])
    w_of = jnp.arange(NW * CAPW, dtype=jnp.int32) // CAPW
    idx = wstart[w_of] + jnp.arange(NW * CAPW, dtype=jnp.int32) % CAPW
    valid = idx < wstart[w_of + 1]
    eidw = jnp.where(valid, jnp.concatenate([order, jnp.zeros((1,), jnp.int32)])[
        jnp.where(valid, idx, E)], E)
    srcw = jnp.where(valid, jnp.concatenate([src, jnp.zeros((1,), jnp.int32)])[eidw], 0)
    dstw = jnp.where(valid, jnp.concatenate([dst, jnp.zeros((1,), jnp.int32)])[eidw], N)
    sdw = jnp.stack([srcw.reshape(NW, NCHUNKW, C), dstw.reshape(NW, NCHUNKW, C),
                     eidw.reshape(NW, NCHUNKW, C)], axis=2)
    ea_pad = jnp.concatenate([edge_attr, jnp.zeros((pad, DE), jnp.float32)], axis=0)
    w_cat = jnp.transpose(W_edge, (1, 0, 2)).reshape(DE, NLAYER * EMB)

    e_all = _edge_emb(ea_pad, w_cat)
    z = jnp.zeros((RPT, EMB), jnp.float32)
    h = _encoder(x, W_node, b_node.reshape(1, EMB))

    for l in range(NLAYER):
        ag = _SC_AGGR[l](h, e_all, sdw, z)
        args = (h, ag, W1[l], b1[l].reshape(1, -1), g1[l].reshape(1, -1),
                be1[l].reshape(1, -1), W2[l], b2[l].reshape(1, -1),
                g2[l].reshape(1, -1), be2[l].reshape(1, -1))
        if l < NLAYER - 1:
            h = _dense(*args)
        else:
            out = _final(*args, batch.reshape(1, N), W_pred,
                         b_pred.reshape(1, NT))
    return out
